# baseline (speedup 1.0000x reference)
import jax
import jax.numpy as jnp
from jax.experimental import pallas as pl

N_USERS = 20000
N_ITEMS = 30000
N_NODES = N_USERS + N_ITEMS
N_INTER = 400000
N_NE = N_NODES + N_INTER
N_LAYERS = 3


def _gcn(x, s, d, W, b, n):
    h = x @ W
    loop = jnp.arange(n, dtype=s.dtype)
    ss = jnp.concatenate([s, loop])
    dd = jnp.concatenate([d, loop])
    deg = jax.ops.segment_sum(jnp.ones(ss.shape[0], dtype=h.dtype), dd, num_segments=n)
    dis = jnp.where(deg > 0, deg ** -0.5, 0.0)
    norm = dis[ss] * dis[dd]
    msg = h[ss] * norm[:, None]
    out = jax.ops.segment_sum(msg, dd, num_segments=n)
    return out + b


def _add_kernel(a_ref, b_ref, o_ref):
    o_ref[...] = a_ref[...] + b_ref[...]


def _padd(a, b):
    n = a.shape[0]
    blk = 2000
    grid = (n + blk - 1) // blk
    return pl.pallas_call(
        _add_kernel,
        grid=(grid,),
        in_specs=[pl.BlockSpec((blk, 64), lambda i: (i, 0)),
                  pl.BlockSpec((blk, 64), lambda i: (i, 0))],
        out_specs=pl.BlockSpec((blk, 64), lambda i: (i, 0)),
        out_shape=jax.ShapeDtypeStruct(a.shape, a.dtype),
    )(a, b)


def kernel(edge_index, node_edge_index, edge_edge_index, edge_features,
           Gu, Gi, Wp, bp, Wnn, bnn, Wee, bee, Wne, bne):
    nn_emb = jnp.concatenate([Gu, Gi], axis=0)
    ee_emb = jnp.tanh(edge_features @ Wp + bp)
    ne_emb = jnp.concatenate([nn_emb, ee_emb], axis=0)
    for l in range(N_LAYERS):
        nn_emb = _gcn(nn_emb, edge_index[0], edge_index[1], Wnn[l], bnn[l], N_NODES)
        ee_emb = _gcn(ee_emb, edge_edge_index[0], edge_edge_index[1], Wee[l], bee[l], N_INTER)
        ne_emb = _gcn(ne_emb, node_edge_index[0], node_edge_index[1], Wne[l], bne[l], N_NE)
        nn_emb = _padd(nn_emb, ne_emb[:N_NODES])
        ee_emb = _padd(ee_emb, ne_emb[N_NODES:])
        ne_emb = jnp.concatenate([nn_emb, ee_emb], axis=0)
    return nn_emb


# speedup vs baseline: 5.9705x; 5.9705x over previous
import functools

import jax
import jax.numpy as jnp
from jax import lax
from jax.experimental import pallas as pl
from jax.experimental.pallas import tpu as pltpu
from jax.experimental.pallas import tpu_sc as plsc

NU, NI = 20000, 30000
N_NODES = NU + NI
N_INTER = 400000
N_NE = N_NODES + N_INTER
E = 800000
EMB = 64
NLAYERS = 3

NC, NS = 2, 16
NW = NC * NS
ESH = E // NW
CHUNK = 128
NCH = (ESH + CHUNK - 1) // CHUNK
NCH_P = 200
ESH_P = NCH_P * CHUNK
CB_LOG2 = 13
CB = 1 << CB_LOG2
EMB_P = 128
BATCH = 128


def _cdiv(a, b):
    return (a + b - 1) // b


def _rup(a, b):
    return _cdiv(a, b) * b


class _G:
    def __init__(self, n):
        self.n = n
        self.nb = _cdiv(n, CB)
        self.nb16 = self.nb * 16
        self.nbp = _rup(self.nb, 16)
        self.s8 = _rup(_cdiv(n, 16), 8)
        self.spd = 16 * self.s8 + 16
        self.ep = E + self.nb * NW * 8 + 256
        self.npad = self.nb * CB


G_NN = _G(N_NODES)
G_EE = _G(N_INTER)
G_NE = _G(N_NE)

_MESH = plsc.VectorSubcoreMesh(core_axis_name="c", subcore_axis_name="s",
                               num_cores=NC, num_subcores=NS)


def _iota16():
    return lax.iota(jnp.int32, 16)


def _make_k1(g):
    @functools.partial(
        pl.kernel,
        out_type=(jax.ShapeDtypeStruct((NW * g.nb16,), jnp.int32),
                  jax.ShapeDtypeStruct((NC * 16 * g.s8,), jnp.float32)),
        mesh=_MESH,
        compiler_params=pltpu.CompilerParams(needs_layout_passes=False),
        scratch_types=dict(
            idx2=pltpu.VMEM((NCH_P, CHUNK), jnp.int32),
            hist=pltpu.VMEM((g.nb16,), jnp.int32),
            ones_v=pltpu.VMEM((CHUNK,), jnp.float32),
            zb=pltpu.VMEM((g.s8,), jnp.float32),
            degsp=pltpu.VMEM_SHARED((g.spd,), jnp.float32),
        ),
    )
    def k1(d_hbm, ones_hbm, zeros_hbm, counts_hbm, degp_hbm,
           idx2, hist, ones_v, zb, degsp):
        c = lax.axis_index("c")
        s = lax.axis_index("s")
        wid = s * NC + c
        it = _iota16()

        pltpu.sync_copy(d_hbm.at[wid], idx2)
        pltpu.sync_copy(ones_hbm, ones_v)
        pltpu.sync_copy(zeros_hbm.at[pl.ds(0, g.s8)], zb)
        pltpu.sync_copy(zb, degsp.at[pl.ds(pl.multiple_of(s * g.s8, 8), g.s8)])

        for i in range(g.nb):
            hist[pl.ds(i * 16, 16)] = jnp.zeros((16,), jnp.int32)

        ones_i = jnp.ones((16,), jnp.int32)
        dump = 16 * g.s8 + (it & 7)

        def chunk_body(ch, _):
            for q in range(CHUNK // 16):
                off = ch * CHUNK + q * 16
                valid = off + it < ESH
                dd = idx2[ch, pl.ds(q * 16, 16)]
                cid = dd >> CB_LOG2
                slot = cid * 16 + it
                plsc.addupdate_scatter(hist, [slot], ones_i, mask=valid)
                idx2[ch, pl.ds(q * 16, 16)] = jnp.where(valid, dd, dump)
            return 0

        lax.fori_loop(0, NCH, chunk_body, 0)
        pltpu.sync_copy(hist, counts_hbm.at[pl.ds(pl.multiple_of(wid * g.nb16, 8), g.nb16)])

        plsc.subcore_barrier()

        def deg_body(ch, _):
            pltpu.sync_copy(ones_v, degsp.at[idx2.at[ch]], add=True)
            return 0

        lax.fori_loop(0, NCH, deg_body, 0)
        plsc.subcore_barrier()
        pltpu.sync_copy(degsp.at[pl.ds(pl.multiple_of(s * g.s8, 8), g.s8)], zb)
        pltpu.sync_copy(zb, degp_hbm.at[pl.ds(pl.multiple_of(c * 16 * g.s8 + s * g.s8, 8), g.s8)])

    return k1


def _make_k2(g):
    @functools.partial(
        pl.kernel,
        out_type=(jax.ShapeDtypeStruct((g.ep,), jnp.int32),
                  jax.ShapeDtypeStruct((g.ep,), jnp.int32)),
        mesh=_MESH,
        compiler_params=pltpu.CompilerParams(needs_layout_passes=False),
        scratch_types=dict(
            sidx2=pltpu.VMEM((NCH_P, CHUNK), jnp.int32),
            didx2=pltpu.VMEM((NCH_P, CHUNK), jnp.int32),
            pos2=pltpu.VMEM((NCH_P, CHUNK), jnp.int32),
            off_v=pltpu.VMEM((g.nb16,), jnp.int32),
        ),
    )
    def k2(s_hbm, d_hbm, poff_hbm, sb_hbm, db_hbm, sidx2, didx2, pos2, off_v):
        c = lax.axis_index("c")
        s = lax.axis_index("s")
        wid = s * NC + c
        it = _iota16()

        pltpu.sync_copy(s_hbm.at[wid], sidx2)
        pltpu.sync_copy(d_hbm.at[wid], didx2)
        pltpu.sync_copy(poff_hbm.at[pl.ds(pl.multiple_of(wid * g.nb16, 8), g.nb16)], off_v)

        dumppos = g.ep - 256 + it

        def chunk_body(ch, _):
            for q in range(CHUNK // 16):
                off = ch * CHUNK + q * 16
                valid = off + it < ESH
                dd = didx2[ch, pl.ds(q * 16, 16)]
                cid = dd >> CB_LOG2
                slot = cid * 16 + it
                cur = plsc.load_gather(off_v, [slot], mask=valid)
                plsc.store_scatter(off_v, [slot], cur + 1, mask=valid)
                pos2[ch, pl.ds(q * 16, 16)] = jnp.where(valid, cur, dumppos)
                didx2[ch, pl.ds(q * 16, 16)] = dd & (CB - 1)
            pltpu.sync_copy(sidx2.at[ch], sb_hbm.at[pos2.at[ch]])
            pltpu.sync_copy(didx2.at[ch], db_hbm.at[pos2.at[ch]])
            return 0

        lax.fori_loop(0, NCH, chunk_body, 0)

    return k2


def _make_k3(g):
    nbp = g.nbp

    @functools.partial(
        pl.kernel,
        out_type=jax.ShapeDtypeStruct((g.npad, EMB_P), jnp.float32),
        mesh=_MESH,
        compiler_params=pltpu.CompilerParams(needs_layout_passes=False),
        scratch_types=dict(
            p2=pltpu.VMEM((2 * nbp,), jnp.int32),
            c2=pltpu.VMEM((2 * nbp,), jnp.int32),
            sidx=pltpu.VMEM((BATCH,), jnp.int32),
            didx=pltpu.VMEM((BATCH,), jnp.int32),
            stage=pltpu.VMEM((BATCH, EMB_P), jnp.float32),
            zv=pltpu.VMEM((128, EMB_P), jnp.float32),
            acc=pltpu.VMEM_SHARED((CB + 16, EMB_P), jnp.float32),
            gsem=pltpu.SemaphoreType.DMA,
        ),
    )
    def k3(hp_hbm, sb_hbm, db_hbm, pst_hbm, cnt_hbm, out_hbm,
           p2, c2, sidx, didx, stage, zv, acc, gsem):
        c = lax.axis_index("c")
        s = lax.axis_index("s")
        it = _iota16()

        pltpu.sync_copy(pst_hbm.at[pl.ds(pl.multiple_of(s * NC * nbp, 8), 2 * nbp)], p2)
        pltpu.sync_copy(cnt_hbm.at[pl.ds(pl.multiple_of(s * NC * nbp, 8), 2 * nbp)], c2)

        def z_body(r, _):
            for q in range(EMB_P // 16):
                zv[r, pl.ds(q * 16, 16)] = jnp.zeros((16,), jnp.float32)
            return 0

        lax.fori_loop(0, 128, z_body, 0)

        def extract(vref, row, col):
            tot = jnp.int32(0)
            for chn in range(nbp // 16):
                v = vref[pl.ds(row * nbp + chn * 16, 16)]
                tot = tot + jnp.sum(jnp.where(it + chn * 16 == col, v, 0))
            return tot

        rows_per_tile = CB // 16

        def bucket_body(bl, _):
            b = bl * NC + c
            def zr(r, _2):
                pltpu.sync_copy(
                    zv, acc.at[pl.ds(s * rows_per_tile + r * 128, 128)])
                return 0
            lax.fori_loop(0, rows_per_tile // 128, zr, 0)
            plsc.subcore_barrier()

            for wk in range(2):
                start = extract(p2, wk, b)
                m = extract(c2, wk, b)
                nbt = (m + BATCH - 1) // BATCH

                def batch_body(j, _2):
                    base = pl.multiple_of(start + j * BATCH, 8)
                    pltpu.sync_copy(sb_hbm.at[pl.ds(base, BATCH)], sidx)
                    pltpu.sync_copy(db_hbm.at[pl.ds(base, BATCH)], didx)
                    v = m - j * BATCH
                    for q in range(BATCH // 16):
                        loc = q * 16 + it
                        valid = loc < v
                        sv = sidx[pl.ds(q * 16, 16)]
                        sidx[pl.ds(q * 16, 16)] = jnp.where(valid, sv, loc * 8)
                        dv = didx[pl.ds(q * 16, 16)]
                        didx[pl.ds(q * 16, 16)] = jnp.where(
                            valid, dv, CB + (it & 7))
                    pltpu.async_copy(hp_hbm.at[sidx], stage, gsem).wait()
                    pltpu.sync_copy(stage, acc.at[didx], add=True)
                    return 0

                lax.fori_loop(0, nbt, batch_body, 0)

            plsc.subcore_barrier()

            def wb(r, _2):
                row = s * rows_per_tile + r * 128
                pltpu.sync_copy(acc.at[pl.ds(row, 128)], stage)
                pltpu.sync_copy(stage, out_hbm.at[pl.ds(b * CB + row, 128)])
                return 0

            lax.fori_loop(0, rows_per_tile // 128, wb, 0)
            return 0

        nbc = (g.nb - c + NC - 1) // NC
        lax.fori_loop(0, nbc, bucket_body, 0)

    return k3


_K1 = {id(g): _make_k1(g) for g in (G_NN, G_EE, G_NE)}
_K2 = {id(g): _make_k2(g) for g in (G_NN, G_EE, G_NE)}
_K3 = {id(g): _make_k3(g) for g in (G_NN, G_EE, G_NE)}


def _mm_dis(x, w, dis_col):
    n, kdim = x.shape
    r = 512
    grid = _cdiv(n, r)

    def body(x_ref, w_ref, d_ref, o_ref):
        xw = jnp.dot(x_ref[...], w_ref[...],
                     preferred_element_type=jnp.float32) * d_ref[...]
        o_ref[...] = jnp.concatenate(
            [xw, jnp.zeros((r, EMB_P - EMB), jnp.float32)], axis=1)

    return pl.pallas_call(
        body,
        grid=(grid,),
        in_specs=[pl.BlockSpec((r, kdim), lambda i: (i, 0)),
                  pl.BlockSpec((kdim, EMB), lambda i: (0, 0)),
                  pl.BlockSpec((r, 1), lambda i: (i, 0))],
        out_specs=pl.BlockSpec((r, EMB_P), lambda i: (i, 0)),
        out_shape=jax.ShapeDtypeStruct((n, EMB_P), jnp.float32),
    )(x, w, dis_col)


def _proj_tanh(x, w, b):
    n, kdim = x.shape
    r = 1024
    grid = _cdiv(n, r)

    def body(x_ref, w_ref, b_ref, o_ref):
        o_ref[...] = jnp.tanh(
            jnp.dot(x_ref[...], w_ref[...],
                    preferred_element_type=jnp.float32) + b_ref[...])

    return pl.pallas_call(
        body,
        grid=(grid,),
        in_specs=[pl.BlockSpec((r, kdim), lambda i: (i, 0)),
                  pl.BlockSpec((kdim, EMB), lambda i: (0, 0)),
                  pl.BlockSpec((1, EMB), lambda i: (0, 0))],
        out_specs=pl.BlockSpec((r, EMB), lambda i: (i, 0)),
        out_shape=jax.ShapeDtypeStruct((n, EMB), jnp.float32),
    )(x, w, b.reshape(1, EMB))


def _tables(g, counts):
    cnt = counts.reshape(NW, g.nb, 16)
    m_bw = cnt.sum(-1).T
    c8 = ((m_bw + 7) // 8) * 8
    flat = c8.reshape(-1)
    starts = (jnp.cumsum(flat) - flat).reshape(g.nb, NW)
    lane_excl = jnp.cumsum(cnt, axis=-1) - cnt
    poff = (starts.T[:, :, None] + lane_excl).reshape(NW * g.nb16)
    padc = ((0, 0), (0, g.nbp - g.nb))
    pst = jnp.pad(starts.T, padc).reshape(NW * g.nbp)
    cm = jnp.pad(m_bw.T, padc).reshape(NW * g.nbp)
    return poff.astype(jnp.int32), pst.astype(jnp.int32), cm.astype(jnp.int32)


def _prep_graph(g, s_arr, d_arr, ones128, zeros_z):
    dpad = jnp.pad(d_arr.reshape(NW, ESH), ((0, 0), (0, ESH_P - ESH))
                   ).reshape(NW, NCH_P, CHUNK)
    spad = jnp.pad(s_arr.reshape(NW, ESH), ((0, 0), (0, ESH_P - ESH))
                   ).reshape(NW, NCH_P, CHUNK)
    counts, degp = _K1[id(g)](dpad, ones128, zeros_z)
    dis = lax.rsqrt(1.0 + degp[:g.n] + degp[16 * g.s8:16 * g.s8 + g.n])
    poff, pst, cm = _tables(g, counts)
    sb, db = _K2[id(g)](spad, dpad, poff)
    return dis, sb, db, pst, cm


def _conv(g, x_parts, w, bvec, prep):
    dis, sb, db, pst, cm = prep
    if len(x_parts) == 1:
        hp = _mm_dis(x_parts[0], w, dis[:, None])
    else:
        n0 = x_parts[0].shape[0]
        hp = jnp.concatenate([
            _mm_dis(x_parts[0], w, dis[:n0, None]),
            _mm_dis(x_parts[1], w, dis[n0:, None])], axis=0)
    acc = _K3[id(g)](hp, sb, db, pst, cm)
    return dis[:, None] * (acc[:g.n, :EMB] + hp[:, :EMB]) + bvec


def kernel(edge_index, node_edge_index, edge_edge_index, edge_features,
           Gu, Gi, Wp, bp, Wnn, bnn, Wee, bee, Wne, bne):
    ones128 = jnp.ones((CHUNK,), jnp.float32)
    zeros_z = jnp.zeros((max(G_NN.s8, G_EE.s8, G_NE.s8),), jnp.float32)

    prep_nn = _prep_graph(G_NN, edge_index[0], edge_index[1], ones128, zeros_z)
    prep_ee = _prep_graph(G_EE, edge_edge_index[0], edge_edge_index[1],
                          ones128, zeros_z)
    prep_ne = _prep_graph(G_NE, node_edge_index[0], node_edge_index[1],
                          ones128, zeros_z)

    nn_emb = jnp.concatenate([Gu, Gi], axis=0)
    ee_emb = _proj_tanh(edge_features, Wp, bp)

    for l in range(NLAYERS):
        nn_c = _conv(G_NN, (nn_emb,), Wnn[l], bnn[l], prep_nn)
        ee_c = _conv(G_EE, (ee_emb,), Wee[l], bee[l], prep_ee)
        ne_c = _conv(G_NE, (nn_emb, ee_emb), Wne[l], bne[l], prep_ne)
        nn_emb = nn_c + ne_c[:N_NODES]
        ee_emb = ee_c + ne_c[N_NODES:]

    return nn_emb


# speedup vs baseline: 6.6381x; 1.1118x over previous
import functools

import jax
import jax.numpy as jnp
from jax import lax
from jax.experimental import pallas as pl
from jax.experimental.pallas import tpu as pltpu
from jax.experimental.pallas import tpu_sc as plsc

NU, NI = 20000, 30000
N_NODES = NU + NI
N_INTER = 400000
N_NE = N_NODES + N_INTER
E = 800000
EMB = 64
NLAYERS = 3

NC, NS = 2, 16
NW = NC * NS
ESH = E // NW
CHUNK = 128
NCH = (ESH + CHUNK - 1) // CHUNK
NCH_P = 200
ESH_P = NCH_P * CHUNK
CB_LOG2 = 13
CB = 1 << CB_LOG2
EMB_P = 128
BATCH = 128


def _cdiv(a, b):
    return (a + b - 1) // b


def _rup(a, b):
    return _cdiv(a, b) * b


class _G:
    def __init__(self, n):
        self.n = n
        self.nb = _cdiv(n, CB)
        self.nb16 = self.nb * 16
        self.nbp = _rup(self.nb, 16)
        self.s8 = _rup(_cdiv(n, 16), 8)
        self.spd = 16 * self.s8 + 16
        self.ep = E + self.nb * NW * 8 + 256
        self.npad = self.nb * CB


G_NN = _G(N_NODES)
G_EE = _G(N_INTER)
G_NE = _G(N_NE)

_MESH = plsc.VectorSubcoreMesh(core_axis_name="c", subcore_axis_name="s",
                               num_cores=NC, num_subcores=NS)


def _iota16():
    return lax.iota(jnp.int32, 16)


def _make_k1(g):
    @functools.partial(
        pl.kernel,
        out_type=(jax.ShapeDtypeStruct((NW * g.nb16,), jnp.int32),
                  jax.ShapeDtypeStruct((NC * 16 * g.s8,), jnp.float32)),
        mesh=_MESH,
        compiler_params=pltpu.CompilerParams(needs_layout_passes=False),
        scratch_types=dict(
            idx2=pltpu.VMEM((NCH_P, CHUNK), jnp.int32),
            hist=pltpu.VMEM((g.nb16,), jnp.int32),
            ones_v=pltpu.VMEM((CHUNK,), jnp.float32),
            zb=pltpu.VMEM((g.s8,), jnp.float32),
            degsp=pltpu.VMEM_SHARED((g.spd,), jnp.float32),
            smA=pltpu.SemaphoreType.DMA,
            smB=pltpu.SemaphoreType.DMA,
        ),
    )
    def k1(d_hbm, ones_hbm, zeros_hbm, counts_hbm, degp_hbm,
           idx2, hist, ones_v, zb, degsp, smA, smB):
        c = lax.axis_index("c")
        s = lax.axis_index("s")
        wid = s * NC + c
        it = _iota16()

        pltpu.sync_copy(d_hbm.at[wid], idx2)
        pltpu.sync_copy(ones_hbm, ones_v)
        pltpu.sync_copy(zeros_hbm.at[pl.ds(0, g.s8)], zb)
        pltpu.sync_copy(zb, degsp.at[pl.ds(pl.multiple_of(s * g.s8, 8), g.s8)])

        for i in range(g.nb):
            hist[pl.ds(i * 16, 16)] = jnp.zeros((16,), jnp.int32)

        ones_i = jnp.ones((16,), jnp.int32)
        dump = 16 * g.s8 + (it & 7)

        def chunk_body(ch, _):
            for q in range(CHUNK // 16):
                off = ch * CHUNK + q * 16
                valid = off + it < ESH
                dd = idx2[ch, pl.ds(q * 16, 16)]
                cid = dd >> CB_LOG2
                slot = cid * 16 + it
                plsc.addupdate_scatter(hist, [slot], ones_i, mask=valid)
                idx2[ch, pl.ds(q * 16, 16)] = jnp.where(valid, dd, dump)
            return 0

        lax.fori_loop(0, NCH, chunk_body, 0)
        pltpu.sync_copy(hist, counts_hbm.at[pl.ds(pl.multiple_of(wid * g.nb16, 8), g.nb16)])

        plsc.subcore_barrier()

        def deg_pair(p, _):
            for half in range(2):
                ch = p * 2 + half
                sem = smA if half == 0 else smB
                pltpu.async_copy(ones_v, degsp.at[idx2.at[ch]], sem, add=True)
                pltpu.make_async_copy(ones_v, degsp.at[idx2.at[ch]], sem).wait()
            return 0

        lax.fori_loop(0, NCH // 2, deg_pair, 0)
        plsc.subcore_barrier()
        pltpu.sync_copy(degsp.at[pl.ds(pl.multiple_of(s * g.s8, 8), g.s8)], zb)
        pltpu.sync_copy(zb, degp_hbm.at[pl.ds(pl.multiple_of(c * 16 * g.s8 + s * g.s8, 8), g.s8)])

    return k1


def _make_k2(g):
    @functools.partial(
        pl.kernel,
        out_type=(jax.ShapeDtypeStruct((g.ep,), jnp.int32),
                  jax.ShapeDtypeStruct((g.ep,), jnp.int32)),
        mesh=_MESH,
        compiler_params=pltpu.CompilerParams(needs_layout_passes=False),
        scratch_types=dict(
            sidx2=pltpu.VMEM((NCH_P, CHUNK), jnp.int32),
            didx2=pltpu.VMEM((NCH_P, CHUNK), jnp.int32),
            pos2=pltpu.VMEM((NCH_P, CHUNK), jnp.int32),
            off_v=pltpu.VMEM((g.nb16,), jnp.int32),
            smA=pltpu.SemaphoreType.DMA,
            smB=pltpu.SemaphoreType.DMA,
        ),
    )
    def k2(s_hbm, d_hbm, poff_hbm, sb_hbm, db_hbm, sidx2, didx2, pos2, off_v,
           smA, smB):
        c = lax.axis_index("c")
        s = lax.axis_index("s")
        wid = s * NC + c
        it = _iota16()

        pltpu.sync_copy(s_hbm.at[wid], sidx2)
        pltpu.sync_copy(d_hbm.at[wid], didx2)
        pltpu.sync_copy(poff_hbm.at[pl.ds(pl.multiple_of(wid * g.nb16, 8), g.nb16)], off_v)

        dumppos = g.ep - 256 + it

        def pair_body(p, _):
            for half in range(2):
                ch = p * 2 + half
                sem = smA if half == 0 else smB
                for q in range(CHUNK // 16):
                    off = ch * CHUNK + q * 16
                    valid = off + it < ESH
                    dd = didx2[ch, pl.ds(q * 16, 16)]
                    cid = dd >> CB_LOG2
                    slot = cid * 16 + it
                    cur = plsc.load_gather(off_v, [slot], mask=valid)
                    plsc.store_scatter(off_v, [slot], cur + 1, mask=valid)
                    pos2[ch, pl.ds(q * 16, 16)] = jnp.where(valid, cur, dumppos)
                    didx2[ch, pl.ds(q * 16, 16)] = dd & (CB - 1)
                pltpu.async_copy(sidx2.at[ch], sb_hbm.at[pos2.at[ch]], sem)
                pltpu.async_copy(didx2.at[ch], db_hbm.at[pos2.at[ch]], sem)
                pltpu.make_async_copy(sidx2.at[ch], sb_hbm.at[pos2.at[ch]], sem).wait()
                pltpu.make_async_copy(didx2.at[ch], db_hbm.at[pos2.at[ch]], sem).wait()
            return 0

        lax.fori_loop(0, NCH // 2, pair_body, 0)

    return k2


def _make_k3(g):
    nbp = g.nbp

    @functools.partial(
        pl.kernel,
        out_type=jax.ShapeDtypeStruct((g.npad, EMB_P), jnp.float32),
        mesh=_MESH,
        compiler_params=pltpu.CompilerParams(needs_layout_passes=False),
        scratch_types=dict(
            p2=pltpu.VMEM((2 * nbp,), jnp.int32),
            c2=pltpu.VMEM((2 * nbp,), jnp.int32),
            sidxA=pltpu.VMEM((BATCH,), jnp.int32),
            didxA=pltpu.VMEM((BATCH,), jnp.int32),
            sidxB=pltpu.VMEM((BATCH,), jnp.int32),
            didxB=pltpu.VMEM((BATCH,), jnp.int32),
            stageA=pltpu.VMEM((BATCH, EMB_P), jnp.float32),
            stageB=pltpu.VMEM((BATCH, EMB_P), jnp.float32),
            zv=pltpu.VMEM((128, EMB_P), jnp.float32),
            acc=pltpu.VMEM_SHARED((CB + 16, EMB_P), jnp.float32),
            si=pltpu.SemaphoreType.DMA,
            sib=pltpu.SemaphoreType.DMA,
            sga=pltpu.SemaphoreType.DMA,
            sgb=pltpu.SemaphoreType.DMA,
            ssa=pltpu.SemaphoreType.DMA,
            ssb=pltpu.SemaphoreType.DMA,
        ),
    )
    def k3(hp_hbm, sb_hbm, db_hbm, pst_hbm, cnt_hbm, out_hbm,
           p2, c2, sidxA, didxA, sidxB, didxB, stageA, stageB, zv, acc,
           si, sib, sga, sgb, ssa, ssb):
        c = lax.axis_index("c")
        s = lax.axis_index("s")
        it = _iota16()

        pltpu.sync_copy(pst_hbm.at[pl.ds(pl.multiple_of(s * NC * nbp, 8), 2 * nbp)], p2)
        pltpu.sync_copy(cnt_hbm.at[pl.ds(pl.multiple_of(s * NC * nbp, 8), 2 * nbp)], c2)

        def z_body(r, _):
            for q in range(EMB_P // 16):
                zv[r, pl.ds(q * 16, 16)] = jnp.zeros((16,), jnp.float32)
            return 0

        lax.fori_loop(0, 128, z_body, 0)

        def extract(vref, row, col):
            tot = jnp.int32(0)
            for chn in range(nbp // 16):
                v = vref[pl.ds(row * nbp + chn * 16, 16)]
                tot = tot + jnp.sum(jnp.where(it + chn * 16 == col, v, 0))
            return tot

        def mask_tail(sref, dref, v):
            for q in range(BATCH // 16):
                loc = q * 16 + it
                valid = loc < v
                sv = sref[pl.ds(q * 16, 16)]
                sref[pl.ds(q * 16, 16)] = jnp.where(valid, sv, loc * 8)
                dv = dref[pl.ds(q * 16, 16)]
                dref[pl.ds(q * 16, 16)] = jnp.where(valid, dv, CB + (it & 7))

        def gwait(sidx, stage, sem):
            pltpu.make_async_copy(hp_hbm.at[sidx], stage, sem).wait()

        def swait(stage, didx, sem):
            pltpu.make_async_copy(stage, acc.at[didx], sem).wait()

        rows_per_tile = CB // 16
        nzc = rows_per_tile // 128

        def bucket_body(bl, _):
            b = bl * NC + c
            for r in range(nzc):
                pltpu.async_copy(
                    zv, acc.at[pl.ds(s * rows_per_tile + r * 128, 128)], si)
            for r in range(nzc):
                pltpu.make_async_copy(
                    zv, acc.at[pl.ds(s * rows_per_tile + r * 128, 128)], si).wait()
            plsc.subcore_barrier()

            for wk in range(2):
                start = extract(p2, wk, b)
                m = extract(c2, wk, b)
                nbt = (m + BATCH - 1) // BATCH
                npair = (nbt + 1) // 2

                def pair_body(p, _2):
                    jA = p * 2
                    jB = jA + 1
                    baseA = pl.multiple_of(start + jA * BATCH, 8)
                    baseB = pl.multiple_of(start + jB * BATCH, 8)
                    hasB = jB < nbt
                    pltpu.async_copy(sb_hbm.at[pl.ds(baseA, BATCH)], sidxA, si)
                    pltpu.async_copy(db_hbm.at[pl.ds(baseA, BATCH)], didxA, si)

                    @pl.when(hasB)
                    def _():
                        pltpu.async_copy(sb_hbm.at[pl.ds(baseB, BATCH)], sidxB, sib)
                        pltpu.async_copy(db_hbm.at[pl.ds(baseB, BATCH)], didxB, sib)

                    pltpu.make_async_copy(sb_hbm.at[pl.ds(baseA, BATCH)], sidxA, si).wait()
                    pltpu.make_async_copy(db_hbm.at[pl.ds(baseA, BATCH)], didxA, si).wait()
                    mask_tail(sidxA, didxA, m - jA * BATCH)
                    pltpu.async_copy(hp_hbm.at[sidxA], stageA, sga)

                    @pl.when(hasB)
                    def _():
                        pltpu.make_async_copy(sb_hbm.at[pl.ds(baseB, BATCH)], sidxB, sib).wait()
                        pltpu.make_async_copy(db_hbm.at[pl.ds(baseB, BATCH)], didxB, sib).wait()
                        mask_tail(sidxB, didxB, m - jB * BATCH)
                        pltpu.async_copy(hp_hbm.at[sidxB], stageB, sgb)

                    gwait(sidxA, stageA, sga)
                    pltpu.async_copy(stageA, acc.at[didxA], ssa, add=True)
                    swait(stageA, didxA, ssa)

                    @pl.when(hasB)
                    def _():
                        gwait(sidxB, stageB, sgb)
                        pltpu.async_copy(stageB, acc.at[didxB], ssb, add=True)
                        swait(stageB, didxB, ssb)

                    return 0

                lax.fori_loop(0, npair, pair_body, 0)

            plsc.subcore_barrier()

            for r in range(nzc):
                st, sm = (stageA, ssa) if r % 2 == 0 else (stageB, ssb)
                row = s * rows_per_tile + r * 128
                if r >= 2:
                    prow = s * rows_per_tile + (r - 2) * 128
                    pltpu.make_async_copy(
                        st, out_hbm.at[pl.ds(b * CB + prow, 128)], sm).wait()
                pltpu.sync_copy(acc.at[pl.ds(row, 128)], st)
                pltpu.async_copy(st, out_hbm.at[pl.ds(b * CB + row, 128)], sm)
            for r in range(2, nzc):
                st, sm = (stageA, ssa) if r % 2 == 0 else (stageB, ssb)
                row = s * rows_per_tile + r * 128
                pltpu.make_async_copy(
                    st, out_hbm.at[pl.ds(b * CB + row, 128)], sm).wait()
            return 0

        nbc = (g.nb - c + NC - 1) // NC
        lax.fori_loop(0, nbc, bucket_body, 0)

    return k3


_K1 = {id(g): _make_k1(g) for g in (G_NN, G_EE, G_NE)}
_K2 = {id(g): _make_k2(g) for g in (G_NN, G_EE, G_NE)}
_K3 = {id(g): _make_k3(g) for g in (G_NN, G_EE, G_NE)}


def _mm_dis(x, w, dis_col):
    n, kdim = x.shape
    r = 512
    grid = _cdiv(n, r)

    def body(x_ref, w_ref, d_ref, o_ref):
        xw = jnp.dot(x_ref[...], w_ref[...],
                     preferred_element_type=jnp.float32) * d_ref[...]
        o_ref[...] = jnp.concatenate(
            [xw, jnp.zeros((r, EMB_P - EMB), jnp.float32)], axis=1)

    return pl.pallas_call(
        body,
        grid=(grid,),
        in_specs=[pl.BlockSpec((r, kdim), lambda i: (i, 0)),
                  pl.BlockSpec((kdim, EMB), lambda i: (0, 0)),
                  pl.BlockSpec((r, 1), lambda i: (i, 0))],
        out_specs=pl.BlockSpec((r, EMB_P), lambda i: (i, 0)),
        out_shape=jax.ShapeDtypeStruct((n, EMB_P), jnp.float32),
    )(x, w, dis_col)


def _proj_tanh(x, w, b):
    n, kdim = x.shape
    r = 1024
    grid = _cdiv(n, r)

    def body(x_ref, w_ref, b_ref, o_ref):
        o_ref[...] = jnp.tanh(
            jnp.dot(x_ref[...], w_ref[...],
                    preferred_element_type=jnp.float32) + b_ref[...])

    return pl.pallas_call(
        body,
        grid=(grid,),
        in_specs=[pl.BlockSpec((r, kdim), lambda i: (i, 0)),
                  pl.BlockSpec((kdim, EMB), lambda i: (0, 0)),
                  pl.BlockSpec((1, EMB), lambda i: (0, 0))],
        out_specs=pl.BlockSpec((r, EMB), lambda i: (i, 0)),
        out_shape=jax.ShapeDtypeStruct((n, EMB), jnp.float32),
    )(x, w, b.reshape(1, EMB))


def _tables(g, counts):
    cnt = counts.reshape(NW, g.nb, 16)
    m_bw = cnt.sum(-1).T
    c8 = ((m_bw + 7) // 8) * 8
    flat = c8.reshape(-1)
    starts = (jnp.cumsum(flat) - flat).reshape(g.nb, NW)
    lane_excl = jnp.cumsum(cnt, axis=-1) - cnt
    poff = (starts.T[:, :, None] + lane_excl).reshape(NW * g.nb16)
    padc = ((0, 0), (0, g.nbp - g.nb))
    pst = jnp.pad(starts.T, padc).reshape(NW * g.nbp)
    cm = jnp.pad(m_bw.T, padc).reshape(NW * g.nbp)
    return poff.astype(jnp.int32), pst.astype(jnp.int32), cm.astype(jnp.int32)


def _prep_graph(g, s_arr, d_arr, ones128, zeros_z):
    dpad = jnp.pad(d_arr.reshape(NW, ESH), ((0, 0), (0, ESH_P - ESH))
                   ).reshape(NW, NCH_P, CHUNK)
    spad = jnp.pad(s_arr.reshape(NW, ESH), ((0, 0), (0, ESH_P - ESH))
                   ).reshape(NW, NCH_P, CHUNK)
    counts, degp = _K1[id(g)](dpad, ones128, zeros_z)
    dis = lax.rsqrt(1.0 + degp[:g.n] + degp[16 * g.s8:16 * g.s8 + g.n])
    poff, pst, cm = _tables(g, counts)
    sb, db = _K2[id(g)](spad, dpad, poff)
    return dis, sb, db, pst, cm


def _conv(g, x_parts, w, bvec, prep):
    dis, sb, db, pst, cm = prep
    if len(x_parts) == 1:
        hp = _mm_dis(x_parts[0], w, dis[:, None])
    else:
        n0 = x_parts[0].shape[0]
        hp = jnp.concatenate([
            _mm_dis(x_parts[0], w, dis[:n0, None]),
            _mm_dis(x_parts[1], w, dis[n0:, None])], axis=0)
    acc = _K3[id(g)](hp, sb, db, pst, cm)
    return dis[:, None] * (acc[:g.n, :EMB] + hp[:, :EMB]) + bvec


def kernel(edge_index, node_edge_index, edge_edge_index, edge_features,
           Gu, Gi, Wp, bp, Wnn, bnn, Wee, bee, Wne, bne):
    ones128 = jnp.ones((CHUNK,), jnp.float32)
    zeros_z = jnp.zeros((max(G_NN.s8, G_EE.s8, G_NE.s8),), jnp.float32)

    prep_nn = _prep_graph(G_NN, edge_index[0], edge_index[1], ones128, zeros_z)
    prep_ee = _prep_graph(G_EE, edge_edge_index[0], edge_edge_index[1],
                          ones128, zeros_z)
    prep_ne = _prep_graph(G_NE, node_edge_index[0], node_edge_index[1],
                          ones128, zeros_z)

    nn_emb = jnp.concatenate([Gu, Gi], axis=0)
    ee_emb = _proj_tanh(edge_features, Wp, bp)

    for l in range(NLAYERS):
        nn_c = _conv(G_NN, (nn_emb,), Wnn[l], bnn[l], prep_nn)
        ee_c = _conv(G_EE, (ee_emb,), Wee[l], bee[l], prep_ee)
        ne_c = _conv(G_NE, (nn_emb, ee_emb), Wne[l], bne[l], prep_ne)
        nn_emb = nn_c + ne_c[:N_NODES]
        ee_emb = ee_c + ne_c[N_NODES:]

    return nn_emb


# speedup vs baseline: 6.6456x; 1.0011x over previous
import functools

import jax
import jax.numpy as jnp
from jax import lax
from jax.experimental import pallas as pl
from jax.experimental.pallas import tpu as pltpu
from jax.experimental.pallas import tpu_sc as plsc

NU, NI = 20000, 30000
N_NODES = NU + NI
N_INTER = 400000
N_NE = N_NODES + N_INTER
E = 800000
EMB = 64
NLAYERS = 3

NC, NS = 2, 16
NW = NC * NS
ESH = E // NW
CHUNK = 128
NCH = (ESH + CHUNK - 1) // CHUNK
NCH_P = 200
ESH_P = NCH_P * CHUNK
CB_LOG2 = 13
CB = 1 << CB_LOG2
EMB_P = 128
BATCH = 128


def _cdiv(a, b):
    return (a + b - 1) // b


def _rup(a, b):
    return _cdiv(a, b) * b


class _G:
    def __init__(self, n):
        self.n = n
        self.nb = _cdiv(n, CB)
        self.nb16 = self.nb * 16
        self.nbp = _rup(self.nb, 16)
        self.s8 = _rup(_cdiv(n, 16), 8)
        self.spd = 16 * self.s8 + 16
        self.ep = E + self.nb * NW * 8 + 256
        self.npad = self.nb * CB


G_NN = _G(N_NODES)
G_EE = _G(N_INTER)
G_NE = _G(N_NE)

_MESH = plsc.VectorSubcoreMesh(core_axis_name="c", subcore_axis_name="s",
                               num_cores=NC, num_subcores=NS)


def _iota16():
    return lax.iota(jnp.int32, 16)


def _make_k1(g):
    @functools.partial(
        pl.kernel,
        out_type=(jax.ShapeDtypeStruct((NW * g.nb16,), jnp.int32),
                  jax.ShapeDtypeStruct((NC * 16 * g.s8,), jnp.float32)),
        mesh=_MESH,
        compiler_params=pltpu.CompilerParams(needs_layout_passes=False),
        scratch_types=dict(
            idx2=pltpu.VMEM((NCH_P, CHUNK), jnp.int32),
            hist=pltpu.VMEM((g.nb16,), jnp.int32),
            ones_v=pltpu.VMEM((CHUNK,), jnp.float32),
            zb=pltpu.VMEM((g.s8,), jnp.float32),
            degsp=pltpu.VMEM_SHARED((g.spd,), jnp.float32),
            smA=pltpu.SemaphoreType.DMA,
            smB=pltpu.SemaphoreType.DMA,
        ),
    )
    def k1(d_hbm, ones_hbm, zeros_hbm, counts_hbm, degp_hbm,
           idx2, hist, ones_v, zb, degsp, smA, smB):
        c = lax.axis_index("c")
        s = lax.axis_index("s")
        wid = s * NC + c
        it = _iota16()

        pltpu.sync_copy(d_hbm.at[wid], idx2)
        pltpu.sync_copy(ones_hbm, ones_v)
        pltpu.sync_copy(zeros_hbm.at[pl.ds(0, g.s8)], zb)
        pltpu.sync_copy(zb, degsp.at[pl.ds(pl.multiple_of(s * g.s8, 8), g.s8)])

        for i in range(g.nb):
            hist[pl.ds(i * 16, 16)] = jnp.zeros((16,), jnp.int32)

        ones_i = jnp.ones((16,), jnp.int32)
        dump = 16 * g.s8 + (it & 7)

        def chunk_body(ch, _):
            for q in range(CHUNK // 16):
                off = ch * CHUNK + q * 16
                valid = off + it < ESH
                dd = idx2[ch, pl.ds(q * 16, 16)]
                cid = dd >> CB_LOG2
                slot = cid * 16 + it
                plsc.addupdate_scatter(hist, [slot], ones_i, mask=valid)
                idx2[ch, pl.ds(q * 16, 16)] = jnp.where(valid, dd, dump)
            return 0

        lax.fori_loop(0, NCH, chunk_body, 0)
        pltpu.sync_copy(hist, counts_hbm.at[pl.ds(pl.multiple_of(wid * g.nb16, 8), g.nb16)])

        plsc.subcore_barrier()

        def deg_pair(p, _):
            for half in range(2):
                ch = p * 2 + half
                sem = smA if half == 0 else smB
                pltpu.async_copy(ones_v, degsp.at[idx2.at[ch]], sem, add=True)
                pltpu.make_async_copy(ones_v, degsp.at[idx2.at[ch]], sem).wait()
            return 0

        lax.fori_loop(0, NCH // 2, deg_pair, 0)
        plsc.subcore_barrier()
        pltpu.sync_copy(degsp.at[pl.ds(pl.multiple_of(s * g.s8, 8), g.s8)], zb)
        pltpu.sync_copy(zb, degp_hbm.at[pl.ds(pl.multiple_of(c * 16 * g.s8 + s * g.s8, 8), g.s8)])

    return k1


def _make_k2(g):
    @functools.partial(
        pl.kernel,
        out_type=(jax.ShapeDtypeStruct((g.ep,), jnp.int32),
                  jax.ShapeDtypeStruct((g.ep,), jnp.int32)),
        mesh=_MESH,
        compiler_params=pltpu.CompilerParams(needs_layout_passes=False),
        scratch_types=dict(
            sidx2=pltpu.VMEM((NCH_P, CHUNK), jnp.int32),
            didx2=pltpu.VMEM((NCH_P, CHUNK), jnp.int32),
            pos2=pltpu.VMEM((NCH_P, CHUNK), jnp.int32),
            off_v=pltpu.VMEM((g.nb16,), jnp.int32),
            smA=pltpu.SemaphoreType.DMA,
            smB=pltpu.SemaphoreType.DMA,
        ),
    )
    def k2(s_hbm, d_hbm, poff_hbm, sb_hbm, db_hbm, sidx2, didx2, pos2, off_v,
           smA, smB):
        c = lax.axis_index("c")
        s = lax.axis_index("s")
        wid = s * NC + c
        it = _iota16()

        pltpu.sync_copy(s_hbm.at[wid], sidx2)
        pltpu.sync_copy(d_hbm.at[wid], didx2)
        pltpu.sync_copy(poff_hbm.at[pl.ds(pl.multiple_of(wid * g.nb16, 8), g.nb16)], off_v)

        dumppos = g.ep - 256 + it

        def pair_body(p, _):
            for half in range(2):
                ch = p * 2 + half
                sem = smA if half == 0 else smB
                for q in range(CHUNK // 16):
                    off = ch * CHUNK + q * 16
                    valid = off + it < ESH
                    dd = didx2[ch, pl.ds(q * 16, 16)]
                    cid = dd >> CB_LOG2
                    slot = cid * 16 + it
                    cur = plsc.load_gather(off_v, [slot], mask=valid)
                    plsc.store_scatter(off_v, [slot], cur + 1, mask=valid)
                    pos2[ch, pl.ds(q * 16, 16)] = jnp.where(valid, cur, dumppos)
                    didx2[ch, pl.ds(q * 16, 16)] = dd & (CB - 1)
                pltpu.async_copy(sidx2.at[ch], sb_hbm.at[pos2.at[ch]], sem)
                pltpu.async_copy(didx2.at[ch], db_hbm.at[pos2.at[ch]], sem)
                pltpu.make_async_copy(sidx2.at[ch], sb_hbm.at[pos2.at[ch]], sem).wait()
                pltpu.make_async_copy(didx2.at[ch], db_hbm.at[pos2.at[ch]], sem).wait()
            return 0

        lax.fori_loop(0, NCH // 2, pair_body, 0)

    return k2


def _make_k3(g):
    nbp = g.nbp

    @functools.partial(
        pl.kernel,
        out_type=jax.ShapeDtypeStruct((g.npad, EMB_P), jnp.float32),
        mesh=_MESH,
        compiler_params=pltpu.CompilerParams(needs_layout_passes=False),
        scratch_types=dict(
            p2=pltpu.VMEM((2 * nbp,), jnp.int32),
            c2=pltpu.VMEM((2 * nbp,), jnp.int32),
            sidxA=pltpu.VMEM((BATCH,), jnp.int32),
            didxA=pltpu.VMEM((BATCH,), jnp.int32),
            sidxB=pltpu.VMEM((BATCH,), jnp.int32),
            didxB=pltpu.VMEM((BATCH,), jnp.int32),
            stageA=pltpu.VMEM((BATCH, EMB_P), jnp.float32),
            stageB=pltpu.VMEM((BATCH, EMB_P), jnp.float32),
            zv=pltpu.VMEM((128, EMB_P), jnp.float32),
            acc=pltpu.VMEM_SHARED((CB + 16, EMB_P), jnp.float32),
            si=pltpu.SemaphoreType.DMA,
            sib=pltpu.SemaphoreType.DMA,
            sga=pltpu.SemaphoreType.DMA,
            sgb=pltpu.SemaphoreType.DMA,
            ssa=pltpu.SemaphoreType.DMA,
            ssb=pltpu.SemaphoreType.DMA,
        ),
    )
    def k3(hp_hbm, sb_hbm, db_hbm, pst_hbm, cnt_hbm, out_hbm,
           p2, c2, sidxA, didxA, sidxB, didxB, stageA, stageB, zv, acc,
           si, sib, sga, sgb, ssa, ssb):
        c = lax.axis_index("c")
        s = lax.axis_index("s")
        it = _iota16()

        pltpu.sync_copy(pst_hbm.at[pl.ds(pl.multiple_of(s * NC * nbp, 8), 2 * nbp)], p2)
        pltpu.sync_copy(cnt_hbm.at[pl.ds(pl.multiple_of(s * NC * nbp, 8), 2 * nbp)], c2)

        def z_body(r, _):
            for q in range(EMB_P // 16):
                zv[r, pl.ds(q * 16, 16)] = jnp.zeros((16,), jnp.float32)
            return 0

        lax.fori_loop(0, 128, z_body, 0)

        def extract(vref, row, col):
            tot = jnp.int32(0)
            for chn in range(nbp // 16):
                v = vref[pl.ds(row * nbp + chn * 16, 16)]
                tot = tot + jnp.sum(jnp.where(it + chn * 16 == col, v, 0))
            return tot

        def mask_tail(sref, dref, v):
            for q in range(BATCH // 16):
                loc = q * 16 + it
                valid = loc < v
                sv = sref[pl.ds(q * 16, 16)]
                sref[pl.ds(q * 16, 16)] = jnp.where(valid, sv, loc * 8)
                dv = dref[pl.ds(q * 16, 16)]
                dref[pl.ds(q * 16, 16)] = jnp.where(valid, dv, CB + (it & 7))

        def gwait(sidx, stage, sem):
            pltpu.make_async_copy(hp_hbm.at[sidx], stage, sem).wait()

        def swait(stage, didx, sem):
            pltpu.make_async_copy(stage, acc.at[didx], sem).wait()

        rows_per_tile = CB // 16
        nzc = rows_per_tile // 128

        def bucket_body(bl, _):
            b = bl * NC + c
            for r in range(nzc):
                pltpu.async_copy(
                    zv, acc.at[pl.ds(s * rows_per_tile + r * 128, 128)], si)
            for r in range(nzc):
                pltpu.make_async_copy(
                    zv, acc.at[pl.ds(s * rows_per_tile + r * 128, 128)], si).wait()
            plsc.subcore_barrier()

            for wk in range(2):
                start = extract(p2, wk, b)
                m = extract(c2, wk, b)
                nbt = (m + BATCH - 1) // BATCH
                npair = (nbt + 1) // 2

                def pair_body(p, _2):
                    jA = p * 2
                    jB = jA + 1
                    baseA = pl.multiple_of(start + jA * BATCH, 8)
                    baseB = pl.multiple_of(start + jB * BATCH, 8)
                    hasB = jB < nbt
                    pltpu.async_copy(sb_hbm.at[pl.ds(baseA, BATCH)], sidxA, si)
                    pltpu.async_copy(db_hbm.at[pl.ds(baseA, BATCH)], didxA, si)

                    @pl.when(hasB)
                    def _():
                        pltpu.async_copy(sb_hbm.at[pl.ds(baseB, BATCH)], sidxB, sib)
                        pltpu.async_copy(db_hbm.at[pl.ds(baseB, BATCH)], didxB, sib)

                    pltpu.make_async_copy(sb_hbm.at[pl.ds(baseA, BATCH)], sidxA, si).wait()
                    pltpu.make_async_copy(db_hbm.at[pl.ds(baseA, BATCH)], didxA, si).wait()
                    mask_tail(sidxA, didxA, m - jA * BATCH)
                    pltpu.async_copy(hp_hbm.at[sidxA], stageA, sga)

                    @pl.when(hasB)
                    def _():
                        pltpu.make_async_copy(sb_hbm.at[pl.ds(baseB, BATCH)], sidxB, sib).wait()
                        pltpu.make_async_copy(db_hbm.at[pl.ds(baseB, BATCH)], didxB, sib).wait()
                        mask_tail(sidxB, didxB, m - jB * BATCH)
                        pltpu.async_copy(hp_hbm.at[sidxB], stageB, sgb)

                    gwait(sidxA, stageA, sga)
                    pltpu.async_copy(stageA, acc.at[didxA], ssa, add=True)

                    @pl.when(hasB)
                    def _():
                        gwait(sidxB, stageB, sgb)
                        pltpu.async_copy(stageB, acc.at[didxB], ssb, add=True)

                    swait(stageA, didxA, ssa)

                    @pl.when(hasB)
                    def _():
                        swait(stageB, didxB, ssb)

                    return 0

                lax.fori_loop(0, npair, pair_body, 0)

            plsc.subcore_barrier()

            for r in range(nzc):
                st, sm = (stageA, ssa) if r % 2 == 0 else (stageB, ssb)
                row = s * rows_per_tile + r * 128
                if r >= 2:
                    prow = s * rows_per_tile + (r - 2) * 128
                    pltpu.make_async_copy(
                        st, out_hbm.at[pl.ds(b * CB + prow, 128)], sm).wait()
                pltpu.sync_copy(acc.at[pl.ds(row, 128)], st)
                pltpu.async_copy(st, out_hbm.at[pl.ds(b * CB + row, 128)], sm)
            for r in range(2, nzc):
                st, sm = (stageA, ssa) if r % 2 == 0 else (stageB, ssb)
                row = s * rows_per_tile + r * 128
                pltpu.make_async_copy(
                    st, out_hbm.at[pl.ds(b * CB + row, 128)], sm).wait()
            return 0

        nbc = (g.nb - c + NC - 1) // NC
        lax.fori_loop(0, nbc, bucket_body, 0)

    return k3


_K1 = {id(g): _make_k1(g) for g in (G_NN, G_EE, G_NE)}
_K2 = {id(g): _make_k2(g) for g in (G_NN, G_EE, G_NE)}
_K3 = {id(g): _make_k3(g) for g in (G_NN, G_EE, G_NE)}


def _mm_dis(x, w, dis_col):
    n, kdim = x.shape
    r = 512
    grid = _cdiv(n, r)

    def body(x_ref, w_ref, d_ref, o_ref):
        xw = jnp.dot(x_ref[...], w_ref[...],
                     preferred_element_type=jnp.float32) * d_ref[...]
        o_ref[...] = jnp.concatenate(
            [xw, jnp.zeros((r, EMB_P - EMB), jnp.float32)], axis=1)

    return pl.pallas_call(
        body,
        grid=(grid,),
        in_specs=[pl.BlockSpec((r, kdim), lambda i: (i, 0)),
                  pl.BlockSpec((kdim, EMB), lambda i: (0, 0)),
                  pl.BlockSpec((r, 1), lambda i: (i, 0))],
        out_specs=pl.BlockSpec((r, EMB_P), lambda i: (i, 0)),
        out_shape=jax.ShapeDtypeStruct((n, EMB_P), jnp.float32),
    )(x, w, dis_col)


def _proj_tanh(x, w, b):
    n, kdim = x.shape
    r = 1024
    grid = _cdiv(n, r)

    def body(x_ref, w_ref, b_ref, o_ref):
        o_ref[...] = jnp.tanh(
            jnp.dot(x_ref[...], w_ref[...],
                    preferred_element_type=jnp.float32) + b_ref[...])

    return pl.pallas_call(
        body,
        grid=(grid,),
        in_specs=[pl.BlockSpec((r, kdim), lambda i: (i, 0)),
                  pl.BlockSpec((kdim, EMB), lambda i: (0, 0)),
                  pl.BlockSpec((1, EMB), lambda i: (0, 0))],
        out_specs=pl.BlockSpec((r, EMB), lambda i: (i, 0)),
        out_shape=jax.ShapeDtypeStruct((n, EMB), jnp.float32),
    )(x, w, b.reshape(1, EMB))


def _tables(g, counts):
    cnt = counts.reshape(NW, g.nb, 16)
    m_bw = cnt.sum(-1).T
    c8 = ((m_bw + 7) // 8) * 8
    flat = c8.reshape(-1)
    starts = (jnp.cumsum(flat) - flat).reshape(g.nb, NW)
    lane_excl = jnp.cumsum(cnt, axis=-1) - cnt
    poff = (starts.T[:, :, None] + lane_excl).reshape(NW * g.nb16)
    padc = ((0, 0), (0, g.nbp - g.nb))
    pst = jnp.pad(starts.T, padc).reshape(NW * g.nbp)
    cm = jnp.pad(m_bw.T, padc).reshape(NW * g.nbp)
    return poff.astype(jnp.int32), pst.astype(jnp.int32), cm.astype(jnp.int32)


def _prep_graph(g, s_arr, d_arr, ones128, zeros_z):
    dpad = jnp.pad(d_arr.reshape(NW, ESH), ((0, 0), (0, ESH_P - ESH))
                   ).reshape(NW, NCH_P, CHUNK)
    spad = jnp.pad(s_arr.reshape(NW, ESH), ((0, 0), (0, ESH_P - ESH))
                   ).reshape(NW, NCH_P, CHUNK)
    counts, degp = _K1[id(g)](dpad, ones128, zeros_z)
    dis = lax.rsqrt(1.0 + degp[:g.n] + degp[16 * g.s8:16 * g.s8 + g.n])
    poff, pst, cm = _tables(g, counts)
    sb, db = _K2[id(g)](spad, dpad, poff)
    return dis, sb, db, pst, cm


def _conv(g, x_parts, w, bvec, prep):
    dis, sb, db, pst, cm = prep
    if len(x_parts) == 1:
        hp = _mm_dis(x_parts[0], w, dis[:, None])
    else:
        n0 = x_parts[0].shape[0]
        hp = jnp.concatenate([
            _mm_dis(x_parts[0], w, dis[:n0, None]),
            _mm_dis(x_parts[1], w, dis[n0:, None])], axis=0)
    acc = _K3[id(g)](hp, sb, db, pst, cm)
    return dis[:, None] * (acc[:g.n, :EMB] + hp[:, :EMB]) + bvec


def kernel(edge_index, node_edge_index, edge_edge_index, edge_features,
           Gu, Gi, Wp, bp, Wnn, bnn, Wee, bee, Wne, bne):
    ones128 = jnp.ones((CHUNK,), jnp.float32)
    zeros_z = jnp.zeros((max(G_NN.s8, G_EE.s8, G_NE.s8),), jnp.float32)

    prep_nn = _prep_graph(G_NN, edge_index[0], edge_index[1], ones128, zeros_z)
    prep_ee = _prep_graph(G_EE, edge_edge_index[0], edge_edge_index[1],
                          ones128, zeros_z)
    prep_ne = _prep_graph(G_NE, node_edge_index[0], node_edge_index[1],
                          ones128, zeros_z)

    nn_emb = jnp.concatenate([Gu, Gi], axis=0)
    ee_emb = _proj_tanh(edge_features, Wp, bp)

    for l in range(NLAYERS):
        nn_c = _conv(G_NN, (nn_emb,), Wnn[l], bnn[l], prep_nn)
        ee_c = _conv(G_EE, (ee_emb,), Wee[l], bee[l], prep_ee)
        ne_c = _conv(G_NE, (nn_emb, ee_emb), Wne[l], bne[l], prep_ne)
        nn_emb = nn_c + ne_c[:N_NODES]
        ee_emb = ee_c + ne_c[N_NODES:]

    return nn_emb
